# baseline (speedup 1.0000x reference)
import jax
import jax.numpy as jnp
from jax.experimental import pallas as pl
from jax.experimental.pallas import tpu as pltpu

N_TOK = 4096
N_CLU = 256
N_FEA = 128
N_CHUNK = 4
CHUNK = N_TOK // N_CHUNK


def _body(x_hbm, c_ref, out_ref, x_vmem, idx_v, idx_s, sems, sem1):
    for j in range(N_CHUNK):
        pltpu.make_async_copy(
            x_hbm.at[pl.ds(j * CHUNK, CHUNK), :],
            x_vmem.at[pl.ds(j * CHUNK, CHUNK), :],
            sems.at[j],
        ).start()
    c = c_ref[:]
    m_run = jnp.full((1, N_CLU), jnp.inf, jnp.float32)
    i_run = jnp.zeros((1, N_CLU), jnp.int32)
    for j in range(N_CHUNK):
        pltpu.make_async_copy(
            x_hbm.at[pl.ds(j * CHUNK, CHUNK), :],
            x_vmem.at[pl.ds(j * CHUNK, CHUNK), :],
            sems.at[j],
        ).wait()
        xj = x_vmem[pl.ds(j * CHUNK, CHUNK), :]
        xnj = jnp.sum(xj * xj, axis=1, keepdims=True)
        xcj = jax.lax.dot_general(
            xj, c, (((1,), (1,)), ((), ())),
            preferred_element_type=jnp.float32,
            precision=jax.lax.Precision.HIGHEST,
        )
        sj = xnj - 2.0 * xcj
        mj = jnp.min(sj, axis=0, keepdims=True)
        rows = jax.lax.broadcasted_iota(jnp.int32, (CHUNK, N_CLU), 0)
        ij = jnp.min(jnp.where(sj == mj, rows + j * CHUNK, N_TOK), axis=0,
                     keepdims=True)
        better = mj < m_run
        m_run = jnp.where(better, mj, m_run)
        i_run = jnp.where(better, ij, i_run)
    idx_v[:, :] = i_run
    cp = pltpu.make_async_copy(idx_v, idx_s, sem1)
    cp.start()
    cp.wait()

    def gather_row(k, carry):
        s = idx_s[0, k]
        out_ref[pl.ds(k, 1), :] = x_vmem[pl.ds(s, 1), :]
        return carry

    jax.lax.fori_loop(0, N_CLU, gather_row, 0, unroll=8)


def kernel(x, cluster_centers):
    x2 = x.reshape(N_TOK, N_FEA)
    out = pl.pallas_call(
        _body,
        out_shape=jax.ShapeDtypeStruct((N_CLU, N_FEA), jnp.float32),
        in_specs=[
            pl.BlockSpec(memory_space=pl.ANY),
            pl.BlockSpec(memory_space=pltpu.VMEM),
        ],
        scratch_shapes=[
            pltpu.VMEM((N_TOK, N_FEA), jnp.float32),
            pltpu.VMEM((1, N_CLU), jnp.int32),
            pltpu.SMEM((1, N_CLU), jnp.int32),
            pltpu.SemaphoreType.DMA((N_CHUNK,)),
            pltpu.SemaphoreType.DMA,
        ],
    )(x2, cluster_centers)
    return out[None]


# speedup vs baseline: 1.4940x; 1.4940x over previous
import jax
import jax.numpy as jnp
from jax.experimental import pallas as pl
from jax.experimental.pallas import tpu as pltpu

N_TOK = 4096
N_CLU = 256
N_FEA = 128
N_BLK = 4
BLK = N_TOK // N_BLK


def _body(x_ref, c_ref, out_ref, x_all, m_run, i_run, idx_v, idx_s, sem):
    j = pl.program_id(0)

    @pl.when(j == 0)
    def _init():
        m_run[:, :] = jnp.full((1, N_CLU), jnp.inf, jnp.float32)
        i_run[:, :] = jnp.zeros((1, N_CLU), jnp.int32)

    xj = x_ref[:]
    x_all[pl.ds(j * BLK, BLK), :] = xj
    c = c_ref[:]
    xnj = jnp.sum(xj * xj, axis=1, keepdims=True)
    xcj = jax.lax.dot_general(
        xj, c, (((1,), (1,)), ((), ())),
        preferred_element_type=jnp.float32,
        precision=jax.lax.Precision.HIGHEST,
    )
    sj = xnj - 2.0 * xcj
    mj = jnp.min(sj, axis=0, keepdims=True)
    rows = jax.lax.broadcasted_iota(jnp.int32, (BLK, N_CLU), 0)
    ij = jnp.min(jnp.where(sj == mj, rows + j * BLK, N_TOK), axis=0,
                 keepdims=True)
    better = mj < m_run[:, :]
    m_run[:, :] = jnp.where(better, mj, m_run[:, :])
    i_run[:, :] = jnp.where(better, ij, i_run[:, :])

    @pl.when(j == N_BLK - 1)
    def _gather():
        idx_v[:, :] = i_run[:, :]
        cp = pltpu.make_async_copy(idx_v, idx_s, sem)
        cp.start()
        cp.wait()

        def gather_row(k, carry):
            s = idx_s[0, k]
            out_ref[pl.ds(k, 1), :] = x_all[pl.ds(s, 1), :]
            return carry

        jax.lax.fori_loop(0, N_CLU, gather_row, 0, unroll=8)


def kernel(x, cluster_centers):
    x2 = x.reshape(N_TOK, N_FEA)
    out = pl.pallas_call(
        _body,
        grid=(N_BLK,),
        out_shape=jax.ShapeDtypeStruct((N_CLU, N_FEA), jnp.float32),
        in_specs=[
            pl.BlockSpec((BLK, N_FEA), lambda j: (j, 0)),
            pl.BlockSpec((N_CLU, N_FEA), lambda j: (0, 0)),
        ],
        out_specs=pl.BlockSpec((N_CLU, N_FEA), lambda j: (0, 0)),
        scratch_shapes=[
            pltpu.VMEM((N_TOK, N_FEA), jnp.float32),
            pltpu.VMEM((1, N_CLU), jnp.float32),
            pltpu.VMEM((1, N_CLU), jnp.int32),
            pltpu.VMEM((1, N_CLU), jnp.int32),
            pltpu.SMEM((1, N_CLU), jnp.int32),
            pltpu.SemaphoreType.DMA,
        ],
    )(x2, cluster_centers)
    return out[None]


# speedup vs baseline: 1.5359x; 1.0280x over previous
import jax
import jax.numpy as jnp
from jax.experimental import pallas as pl
from jax.experimental.pallas import tpu as pltpu

N_TOK = 4096
N_CLU = 256
N_FEA = 128


def _body(x_ref, c_ref, out_ref, idx_v, idx_s, sem):
    x = x_ref[:]
    c = c_ref[:]
    xn = jnp.sum(x * x, axis=1, keepdims=True)
    xc = jax.lax.dot_general(
        x, c, (((1,), (1,)), ((), ())),
        preferred_element_type=jnp.float32,
        precision=jax.lax.Precision.HIGHEST,
    )
    scores = xn - 2.0 * xc
    m = jnp.min(scores, axis=0, keepdims=True)
    rows = jax.lax.broadcasted_iota(jnp.int32, (N_TOK, N_CLU), 0)
    idx_v[0, :] = jnp.min(jnp.where(scores == m, rows, N_TOK), axis=0)
    copy = pltpu.make_async_copy(idx_v, idx_s, sem)
    copy.start()
    copy.wait()

    def gather_row(k, carry):
        s = idx_s[0, k]
        out_ref[pl.ds(k, 1), :] = x_ref[pl.ds(s, 1), :]
        return carry

    jax.lax.fori_loop(0, N_CLU, gather_row, 0, unroll=8)


def kernel(x, cluster_centers):
    x2 = x.reshape(N_TOK, N_FEA)
    out = pl.pallas_call(
        _body,
        out_shape=jax.ShapeDtypeStruct((N_CLU, N_FEA), jnp.float32),
        scratch_shapes=[
            pltpu.VMEM((1, N_CLU), jnp.int32),
            pltpu.SMEM((1, N_CLU), jnp.int32),
            pltpu.SemaphoreType.DMA,
        ],
    )(x2, cluster_centers)
    return out[None]


# speedup vs baseline: 1.7097x; 1.1132x over previous
import jax
import jax.numpy as jnp
from jax.experimental import pallas as pl
from jax.experimental.pallas import tpu as pltpu

N_TOK = 4096
N_CLU = 256
N_FEA = 128


def _body(x_ref, c_ref, out_ref, idx_v, idx_s, sem):
    x = x_ref[:]
    c = c_ref[:]
    xn = jnp.sum(x * x, axis=1, keepdims=True)
    xc = jax.lax.dot_general(
        x, c, (((1,), (1,)), ((), ())),
        preferred_element_type=jnp.float32,
        precision=jax.lax.Precision.HIGHEST,
    )
    scores = xn - 2.0 * xc
    idx_v[0, :] = jnp.argmin(scores, axis=0).astype(jnp.int32)
    copy = pltpu.make_async_copy(idx_v, idx_s, sem)
    copy.start()
    copy.wait()

    def gather_row(k, carry):
        s = idx_s[0, k]
        out_ref[pl.ds(k, 1), :] = x_ref[pl.ds(s, 1), :]
        return carry

    jax.lax.fori_loop(0, N_CLU, gather_row, 0, unroll=16)


def kernel(x, cluster_centers):
    x2 = x.reshape(N_TOK, N_FEA)
    out = pl.pallas_call(
        _body,
        out_shape=jax.ShapeDtypeStruct((N_CLU, N_FEA), jnp.float32),
        scratch_shapes=[
            pltpu.VMEM((1, N_CLU), jnp.int32),
            pltpu.SMEM((1, N_CLU), jnp.int32),
            pltpu.SemaphoreType.DMA,
        ],
    )(x2, cluster_centers)
    return out[None]
